# Initial kernel scaffold; baseline (speedup 1.0000x reference)
import jax
import jax.numpy as jnp
from jax.experimental import pallas as pl


def kernel(h, distances, edges, node_mask, edge_mask, W_lin, bias, ln_gamma, ln_beta, att_W1, att_b1, att_W2, att_b2):
    raise NotImplementedError("write your pallas kernel here")


# speedup vs baseline: 3.1369x; 3.1369x over previous
import functools

import jax
import jax.numpy as jnp
from jax import lax
from jax.experimental import pallas as pl
from jax.experimental.pallas import tpu as pltpu
from jax.experimental.pallas import tpu_sc as plsc

N = 10000
E = 320000
D = 128
MIN = 1e-15
MAXNORM = 1.0 - 4e-3

NC = 2
NS = 16
NW = NC * NS
EPW = E // NW
CH = 80
NCHUNK = EPW // CH

BN = 1000
BE = 2000


def _artanh(x):
    x = jnp.clip(x, -1.0 + 1e-7, 1.0 - 1e-7)
    return 0.5 * jnp.log((1.0 + x) / (1.0 - x))


def _rnorm(x):
    return jnp.maximum(jnp.sqrt(jnp.sum(x * x, axis=-1, keepdims=True)), MIN)


def _proj(x):
    n = _rnorm(x)
    return jnp.where(n > MAXNORM, x / n * MAXNORM, x)


def _expmap0(u):
    un = _rnorm(u)
    return _proj(jnp.tanh(un) * u / un)


def _logmap0(p):
    pn = _rnorm(p)
    return p / pn * _artanh(pn)


def _mobius_add(x, y):
    x2 = jnp.sum(x * x, axis=-1, keepdims=True)
    y2 = jnp.sum(y * y, axis=-1, keepdims=True)
    xy = jnp.sum(x * y, axis=-1, keepdims=True)
    num = (1.0 + 2.0 * xy + y2) * x + (1.0 - x2) * y
    den = 1.0 + 2.0 * xy + x2 * y2
    return num / jnp.maximum(den, MIN)


def _dotT(a, b):
    return lax.dot_general(a, b, (((1,), (1,)), ((), ())),
                           preferred_element_type=jnp.float32)


def _tc1_body(h_ref, wlin_ref, bias_ref, w1a_ref, w1b_ref, b1_ref,
              x_ref, a_ref, b_ref):
    h = h_ref[...]
    x1 = _logmap0(h)
    xw = _dotT(x1, wlin_ref[...])
    xe = _expmap0(xw)
    hb = _expmap0(bias_ref[...])
    x = _mobius_add(xe, hb)
    x_ref[...] = x
    x_t = _logmap0(x)
    a_ref[...] = _dotT(x_t, w1a_ref[...]) + b1_ref[...]
    b_ref[...] = _dotT(x_t, w1b_ref[...])


def _tc1_call(h, W_lin, bias, W1a, W1b, b1):
    nspec = pl.BlockSpec((BN, D), lambda i: (i, 0))
    wspec = pl.BlockSpec((D, D), lambda i: (0, 0))
    vspec = pl.BlockSpec((1, D), lambda i: (0, 0))
    return pl.pallas_call(
        _tc1_body,
        grid=(N // BN,),
        in_specs=[nspec, wspec, vspec, wspec, wspec, vspec],
        out_specs=[nspec, nspec, nspec],
        out_shape=[jax.ShapeDtypeStruct((N, D), jnp.float32)] * 3,
    )(h, W_lin, bias, W1a, W1b, b1)


def _sc1_body(x_hbm, a_hbm, b_hbm, row_hbm, col_hbm,
              xr_hbm, xc_hbm, ar_hbm, bc_hbm,
              rowv, colv, bx1, bx2, ba, bb, sem):
    wid = lax.axis_index("s") * NC + lax.axis_index("c")
    base = wid * EPW

    def body(k, carry):
        off = base + k * CH
        pltpu.sync_copy(row_hbm.at[pl.ds(off, CH)], rowv)
        pltpu.sync_copy(col_hbm.at[pl.ds(off, CH)], colv)
        c1 = pltpu.async_copy(x_hbm.at[rowv], bx1, sem)
        c2 = pltpu.async_copy(x_hbm.at[colv], bx2, sem)
        c3 = pltpu.async_copy(a_hbm.at[rowv], ba, sem)
        c4 = pltpu.async_copy(b_hbm.at[colv], bb, sem)
        c1.wait()
        c2.wait()
        c3.wait()
        c4.wait()
        pltpu.sync_copy(bx1, xr_hbm.at[pl.ds(off, CH)])
        pltpu.sync_copy(bx2, xc_hbm.at[pl.ds(off, CH)])
        pltpu.sync_copy(ba, ar_hbm.at[pl.ds(off, CH)])
        pltpu.sync_copy(bb, bc_hbm.at[pl.ds(off, CH)])
        return carry

    lax.fori_loop(0, NCHUNK, body, 0)


def _sc1_call(x, A, B, row, col):
    mesh = plsc.VectorSubcoreMesh(core_axis_name="c", subcore_axis_name="s")
    f = pl.kernel(
        _sc1_body,
        mesh=mesh,
        out_type=[jax.ShapeDtypeStruct((E, D), jnp.float32)] * 4,
        scratch_types=[
            pltpu.VMEM((CH,), jnp.int32),
            pltpu.VMEM((CH,), jnp.int32),
            pltpu.VMEM((CH, D), jnp.float32),
            pltpu.VMEM((CH, D), jnp.float32),
            pltpu.VMEM((CH, D), jnp.float32),
            pltpu.VMEM((CH, D), jnp.float32),
            pltpu.SemaphoreType.DMA,
        ],
    )
    return f(x, A, B, row, col)


def _tc2_body(xr_ref, xc_ref, ar_ref, bc_ref, d_ref, em_ref,
              u_ref, v_ref, w2_ref, b2_ref, agg_ref):
    p1 = xr_ref[...]
    p2 = xc_ref[...]
    x2 = jnp.sum(p1 * p1, axis=-1, keepdims=True)
    y2 = jnp.sum(p2 * p2, axis=-1, keepdims=True)
    dot = jnp.sum(p1 * p2, axis=-1, keepdims=True)
    alpha = 1.0 - 2.0 * dot + y2
    beta = 1.0 - x2
    num2 = alpha * alpha * x2 + beta * beta * y2 - 2.0 * alpha * beta * dot
    den = jnp.maximum(1.0 - 2.0 * dot + x2 * y2, MIN)
    subn = jnp.maximum(jnp.sqrt(jnp.maximum(num2, 0.0)) / den, MIN)
    w = _artanh(subn)
    geo = 4.0 * w * w
    z = ar_ref[...] + bc_ref[...] + d_ref[...] * u_ref[...] + geo * v_ref[...]
    mid = z * jax.nn.sigmoid(z)
    s = jnp.sum(mid * w2_ref[...], axis=-1, keepdims=True) + b2_ref[0, 0]
    att = jax.nn.sigmoid(s) * em_ref[...]
    scal = att * jnp.maximum(beta, MIN) * w / (den * subn) * 0.01
    agg_ref[...] = scal * (beta * p2 - alpha * p1)


def _tc2_call(xr, xc, ar, bc, distances, edge_mask, u, v, w2, b2):
    espec = pl.BlockSpec((BE, D), lambda i: (i, 0))
    sspec = pl.BlockSpec((BE, 1), lambda i: (i, 0))
    vspec = pl.BlockSpec((1, D), lambda i: (0, 0))
    return pl.pallas_call(
        _tc2_body,
        grid=(E // BE,),
        in_specs=[espec, espec, espec, espec, sspec, sspec,
                  vspec, vspec, vspec, vspec],
        out_specs=espec,
        out_shape=jax.ShapeDtypeStruct((E, D), jnp.float32),
    )(xr, xc, ar, bc, distances, edge_mask, u, v, w2, b2)


def _sc2_body(agg_hbm, row_hbm, zero_hbm, out_hbm,
              rowv, buf, acc, sem):
    cid = lax.axis_index("c")
    sid = lax.axis_index("s")
    wid = sid * NC + cid
    rows = 624
    pltpu.sync_copy(zero_hbm.at[pl.ds(sid * rows, rows)],
                    acc.at[pl.ds(sid * rows, rows)])

    @pl.when(sid == 0)
    def _():
        pltpu.sync_copy(zero_hbm.at[pl.ds(NS * rows, N - NS * rows)],
                        acc.at[pl.ds(NS * rows, N - NS * rows)])

    plsc.subcore_barrier()

    base = wid * EPW

    def body(k, carry):
        off = base + k * CH
        pltpu.sync_copy(row_hbm.at[pl.ds(off, CH)], rowv)
        pltpu.async_copy(agg_hbm.at[pl.ds(off, CH)], buf, sem).wait()
        pltpu.sync_copy(buf, acc.at[rowv], add=True)
        return carry

    lax.fori_loop(0, NCHUNK, body, 0)

    plsc.subcore_barrier()

    @pl.when(sid == 0)
    def _():
        pltpu.sync_copy(acc, out_hbm.at[cid])


def _sc2_call(agg, row, zero):
    mesh = plsc.VectorSubcoreMesh(core_axis_name="c", subcore_axis_name="s")
    f = pl.kernel(
        _sc2_body,
        mesh=mesh,
        out_type=jax.ShapeDtypeStruct((NC, N, D), jnp.float32),
        scratch_types=[
            pltpu.VMEM((CH,), jnp.int32),
            pltpu.VMEM((CH, D), jnp.float32),
            pltpu.VMEM_SHARED((N, D), jnp.float32),
            pltpu.SemaphoreType.DMA,
        ],
    )
    return f(agg, row, zero)


def _tc4_body(p_ref, x_ref, g_ref, bta_ref, o_ref):
    out = p_ref[0] + p_ref[1]
    x = x_ref[...]
    un = _rnorm(out)
    x2 = jnp.sum(x * x, axis=-1, keepdims=True)
    lam = 2.0 / jnp.maximum(1.0 - x2, MIN)
    second = jnp.tanh(0.5 * lam * un) * out / un
    x1 = _proj(_mobius_add(x, second))
    t = _logmap0(x1)
    mu = jnp.mean(t, axis=-1, keepdims=True)
    var = jnp.mean((t - mu) ** 2, axis=-1, keepdims=True)
    t = (t - mu) / jnp.sqrt(var + 1e-5) * g_ref[...] + bta_ref[...]
    x2e = _expmap0(t)
    xt = _logmap0(x2e)
    xt = xt * jax.nn.sigmoid(xt)
    o_ref[...] = _expmap0(xt)


def _tc4_call(parts, x, ln_gamma, ln_beta):
    nspec = pl.BlockSpec((BN, D), lambda i: (i, 0))
    pspec = pl.BlockSpec((NC, BN, D), lambda i: (0, i, 0))
    vspec = pl.BlockSpec((1, D), lambda i: (0, 0))
    return pl.pallas_call(
        _tc4_body,
        grid=(N // BN,),
        in_specs=[pspec, nspec, vspec, vspec],
        out_specs=nspec,
        out_shape=jax.ShapeDtypeStruct((N, D), jnp.float32),
    )(parts, x, ln_gamma, ln_beta)


def kernel(h, distances, edges, node_mask, edge_mask, W_lin, bias,
           ln_gamma, ln_beta, att_W1, att_b1, att_W2, att_b2):
    row = edges[0]
    col = edges[1]
    W1a = att_W1[:, :D]
    W1b = att_W1[:, D:2 * D]
    u = att_W1[:, 2 * D].reshape(1, D)
    v = att_W1[:, 2 * D + 1].reshape(1, D)
    w2 = att_W2.reshape(1, D)
    b1 = att_b1.reshape(1, D)
    b2 = jnp.broadcast_to(att_b2.reshape(1, 1), (1, D))

    x, A, B = _tc1_call(h, W_lin, bias, W1a, W1b, b1)
    xr, xc, ar, bc = _sc1_call(x, A, B, row, col)
    agg = _tc2_call(xr, xc, ar, bc, distances, edge_mask, u, v, w2, b2)
    zero = jnp.zeros((N, D), jnp.float32)
    parts = _sc2_call(agg, row, zero)
    xf = _tc4_call(parts, x, ln_gamma.reshape(1, D), ln_beta.reshape(1, D))
    return (xf, distances, edges, node_mask, edge_mask)


# speedup vs baseline: 3.7971x; 1.2105x over previous
import functools

import jax
import jax.numpy as jnp
from jax import lax
from jax.experimental import pallas as pl
from jax.experimental.pallas import tpu as pltpu
from jax.experimental.pallas import tpu_sc as plsc

N = 10000
E = 320000
D = 128
MIN = 1e-15
MAXNORM = 1.0 - 4e-3

NC = 2
NS = 16
NW = NC * NS
EPW = E // NW
CH = 80
NCHUNK = EPW // CH

BN = 1000
BE = 2000


def _artanh(x):
    x = jnp.clip(x, -1.0 + 1e-7, 1.0 - 1e-7)
    return 0.5 * jnp.log((1.0 + x) / (1.0 - x))


def _rnorm(x):
    return jnp.maximum(jnp.sqrt(jnp.sum(x * x, axis=-1, keepdims=True)), MIN)


def _proj(x):
    n = _rnorm(x)
    return jnp.where(n > MAXNORM, x / n * MAXNORM, x)


def _expmap0(u):
    un = _rnorm(u)
    return _proj(jnp.tanh(un) * u / un)


def _logmap0(p):
    pn = _rnorm(p)
    return p / pn * _artanh(pn)


def _mobius_add(x, y):
    x2 = jnp.sum(x * x, axis=-1, keepdims=True)
    y2 = jnp.sum(y * y, axis=-1, keepdims=True)
    xy = jnp.sum(x * y, axis=-1, keepdims=True)
    num = (1.0 + 2.0 * xy + y2) * x + (1.0 - x2) * y
    den = 1.0 + 2.0 * xy + x2 * y2
    return num / jnp.maximum(den, MIN)


def _dotT(a, b):
    return lax.dot_general(a, b, (((1,), (1,)), ((), ())),
                           preferred_element_type=jnp.float32)


def _tc1_body(h_ref, wlin_ref, bias_ref, w1a_ref, w1b_ref, b1_ref,
              x_ref, a_ref, b_ref):
    h = h_ref[...]
    x1 = _logmap0(h)
    xw = _dotT(x1, wlin_ref[...])
    xe = _expmap0(xw)
    hb = _expmap0(bias_ref[...])
    x = _mobius_add(xe, hb)
    x_ref[...] = x
    x_t = _logmap0(x)
    a_ref[...] = _dotT(x_t, w1a_ref[...]) + b1_ref[...]
    b_ref[...] = _dotT(x_t, w1b_ref[...])


def _tc1_call(h, W_lin, bias, W1a, W1b, b1):
    nspec = pl.BlockSpec((BN, D), lambda i: (i, 0))
    wspec = pl.BlockSpec((D, D), lambda i: (0, 0))
    vspec = pl.BlockSpec((1, D), lambda i: (0, 0))
    return pl.pallas_call(
        _tc1_body,
        grid=(N // BN,),
        in_specs=[nspec, wspec, vspec, wspec, wspec, vspec],
        out_specs=[nspec, nspec, nspec],
        out_shape=[jax.ShapeDtypeStruct((N, D), jnp.float32)] * 3,
    )(h, W_lin, bias, W1a, W1b, b1)


def _sc1_body(x_hbm, a_hbm, b_hbm, row_hbm, col_hbm,
              xr_hbm, xc_hbm, ar_hbm, bc_hbm,
              rowv, colv,
              b00, b01, b02, b03, b10, b11, b12, b13, sem0, sem1):
    bufs = ((b00, b01, b02, b03), (b10, b11, b12, b13))
    sems = (sem0, sem1)
    wid = lax.axis_index("s") * NC + lax.axis_index("c")
    base = wid * EPW
    pltpu.sync_copy(row_hbm.at[pl.ds(base, EPW)], rowv)
    pltpu.sync_copy(col_hbm.at[pl.ds(base, EPW)], colv)

    def fire(k, s):
        o = k * CH
        ri = rowv.at[pl.ds(o, CH)]
        ci = colv.at[pl.ds(o, CH)]
        pltpu.async_copy(x_hbm.at[ri], bufs[s][0], sems[s])
        pltpu.async_copy(x_hbm.at[ci], bufs[s][1], sems[s])
        pltpu.async_copy(a_hbm.at[ri], bufs[s][2], sems[s])
        pltpu.async_copy(b_hbm.at[ci], bufs[s][3], sems[s])

    def drain(k, s):
        for b in range(4):
            pltpu.make_async_copy(x_hbm.at[pl.ds(0, CH)], bufs[s][b],
                                  sems[s]).wait()
        off = base + k * CH
        pltpu.sync_copy(bufs[s][0], xr_hbm.at[pl.ds(off, CH)])
        pltpu.sync_copy(bufs[s][1], xc_hbm.at[pl.ds(off, CH)])
        pltpu.sync_copy(bufs[s][2], ar_hbm.at[pl.ds(off, CH)])
        pltpu.sync_copy(bufs[s][3], bc_hbm.at[pl.ds(off, CH)])

    fire(0, 0)

    def body(j, carry):
        k0 = 2 * j
        fire(k0 + 1, 1)
        drain(k0, 0)
        fire(k0 + 2, 0)
        drain(k0 + 1, 1)
        return carry

    lax.fori_loop(0, (NCHUNK - 1) // 2, body, 0)
    drain(NCHUNK - 1, 0)


def _sc1_call(x, A, B, row, col):
    mesh = plsc.VectorSubcoreMesh(core_axis_name="c", subcore_axis_name="s")
    f = pl.kernel(
        _sc1_body,
        mesh=mesh,
        out_type=[jax.ShapeDtypeStruct((E, D), jnp.float32)] * 4,
        scratch_types=[pltpu.VMEM((EPW,), jnp.int32),
                       pltpu.VMEM((EPW,), jnp.int32)]
                      + [pltpu.VMEM((CH, D), jnp.float32)] * 8
                      + [pltpu.SemaphoreType.DMA] * 2,
    )
    return f(x, A, B, row, col)


def _tc2_body(xr_ref, xc_ref, ar_ref, bc_ref, d_ref, em_ref,
              u_ref, v_ref, w2_ref, b2_ref, agg_ref):
    p1 = xr_ref[...]
    p2 = xc_ref[...]
    x2 = jnp.sum(p1 * p1, axis=-1, keepdims=True)
    y2 = jnp.sum(p2 * p2, axis=-1, keepdims=True)
    dot = jnp.sum(p1 * p2, axis=-1, keepdims=True)
    alpha = 1.0 - 2.0 * dot + y2
    beta = 1.0 - x2
    num2 = alpha * alpha * x2 + beta * beta * y2 - 2.0 * alpha * beta * dot
    den = jnp.maximum(1.0 - 2.0 * dot + x2 * y2, MIN)
    subn = jnp.maximum(jnp.sqrt(jnp.maximum(num2, 0.0)) / den, MIN)
    w = _artanh(subn)
    geo = 4.0 * w * w
    z = ar_ref[...] + bc_ref[...] + d_ref[...] * u_ref[...] + geo * v_ref[...]
    mid = z * jax.nn.sigmoid(z)
    s = jnp.sum(mid * w2_ref[...], axis=-1, keepdims=True) + b2_ref[0, 0]
    att = jax.nn.sigmoid(s) * em_ref[...]
    scal = att * jnp.maximum(beta, MIN) * w / (den * subn) * 0.01
    agg_ref[...] = scal * (beta * p2 - alpha * p1)


def _tc2_call(xr, xc, ar, bc, distances, edge_mask, u, v, w2, b2):
    espec = pl.BlockSpec((BE, D), lambda i: (i, 0))
    sspec = pl.BlockSpec((BE, 1), lambda i: (i, 0))
    vspec = pl.BlockSpec((1, D), lambda i: (0, 0))
    return pl.pallas_call(
        _tc2_body,
        grid=(E // BE,),
        in_specs=[espec, espec, espec, espec, sspec, sspec,
                  vspec, vspec, vspec, vspec],
        out_specs=espec,
        out_shape=jax.ShapeDtypeStruct((E, D), jnp.float32),
    )(xr, xc, ar, bc, distances, edge_mask, u, v, w2, b2)


def _sc2_body(agg_hbm, row3_hbm, zero_hbm, out_hbm,
              rowv2, buf0, buf1, acc, sem0, sem1):
    cid = lax.axis_index("c")
    sid = lax.axis_index("s")
    wid = sid * NC + cid
    rows = 624
    pltpu.sync_copy(zero_hbm.at[pl.ds(sid * rows, rows)],
                    acc.at[pl.ds(sid * rows, rows)])

    @pl.when(sid == 0)
    def _():
        pltpu.sync_copy(zero_hbm.at[pl.ds(NS * rows, N - NS * rows)],
                        acc.at[pl.ds(NS * rows, N - NS * rows)])

    pltpu.sync_copy(row3_hbm.at[wid], rowv2)
    plsc.subcore_barrier()

    base = wid * EPW
    bufs = (buf0, buf1)
    sems = (sem0, sem1)

    def fire(k, s):
        pltpu.async_copy(agg_hbm.at[pl.ds(base + k * CH, CH)], bufs[s], sems[s])

    def drain(k, s):
        pltpu.make_async_copy(agg_hbm.at[pl.ds(0, CH)], bufs[s], sems[s]).wait()
        pltpu.sync_copy(bufs[s], acc.at[rowv2.at[k]], add=True)

    fire(0, 0)

    def body(j, carry):
        k0 = 2 * j
        fire(k0 + 1, 1)
        drain(k0, 0)
        fire(k0 + 2, 0)
        drain(k0 + 1, 1)
        return carry

    lax.fori_loop(0, (NCHUNK - 1) // 2, body, 0)
    drain(NCHUNK - 1, 0)

    plsc.subcore_barrier()

    @pl.when(sid == 0)
    def _():
        pltpu.sync_copy(acc, out_hbm.at[cid])


def _sc2_call(agg, row3, zero):
    mesh = plsc.VectorSubcoreMesh(core_axis_name="c", subcore_axis_name="s")
    f = pl.kernel(
        _sc2_body,
        mesh=mesh,
        out_type=jax.ShapeDtypeStruct((NC, N, D), jnp.float32),
        scratch_types=[
            pltpu.VMEM((NCHUNK, CH), jnp.int32),
            pltpu.VMEM((CH, D), jnp.float32),
            pltpu.VMEM((CH, D), jnp.float32),
            pltpu.VMEM_SHARED((N, D), jnp.float32),
            pltpu.SemaphoreType.DMA,
            pltpu.SemaphoreType.DMA,
        ],
    )
    return f(agg, row3, zero)


def _tc4_body(p_ref, x_ref, g_ref, bta_ref, o_ref):
    out = p_ref[0] + p_ref[1]
    x = x_ref[...]
    un = _rnorm(out)
    x2 = jnp.sum(x * x, axis=-1, keepdims=True)
    lam = 2.0 / jnp.maximum(1.0 - x2, MIN)
    second = jnp.tanh(0.5 * lam * un) * out / un
    x1 = _proj(_mobius_add(x, second))
    t = _logmap0(x1)
    mu = jnp.mean(t, axis=-1, keepdims=True)
    var = jnp.mean((t - mu) ** 2, axis=-1, keepdims=True)
    t = (t - mu) / jnp.sqrt(var + 1e-5) * g_ref[...] + bta_ref[...]
    x2e = _expmap0(t)
    xt = _logmap0(x2e)
    xt = xt * jax.nn.sigmoid(xt)
    o_ref[...] = _expmap0(xt)


def _tc4_call(parts, x, ln_gamma, ln_beta):
    nspec = pl.BlockSpec((BN, D), lambda i: (i, 0))
    pspec = pl.BlockSpec((NC, BN, D), lambda i: (0, i, 0))
    vspec = pl.BlockSpec((1, D), lambda i: (0, 0))
    return pl.pallas_call(
        _tc4_body,
        grid=(N // BN,),
        in_specs=[pspec, nspec, vspec, vspec],
        out_specs=nspec,
        out_shape=jax.ShapeDtypeStruct((N, D), jnp.float32),
    )(parts, x, ln_gamma, ln_beta)


def kernel(h, distances, edges, node_mask, edge_mask, W_lin, bias,
           ln_gamma, ln_beta, att_W1, att_b1, att_W2, att_b2):
    row = edges[0]
    col = edges[1]
    W1a = att_W1[:, :D]
    W1b = att_W1[:, D:2 * D]
    u = att_W1[:, 2 * D].reshape(1, D)
    v = att_W1[:, 2 * D + 1].reshape(1, D)
    w2 = att_W2.reshape(1, D)
    b1 = att_b1.reshape(1, D)
    b2 = jnp.broadcast_to(att_b2.reshape(1, 1), (1, D))

    x, A, B = _tc1_call(h, W_lin, bias, W1a, W1b, b1)
    xr, xc, ar, bc = _sc1_call(x, A, B, row, col)
    agg = _tc2_call(xr, xc, ar, bc, distances, edge_mask, u, v, w2, b2)
    zero = jnp.zeros((N, D), jnp.float32)
    parts = _sc2_call(agg, row.reshape(NW, NCHUNK, CH), zero)
    xf = _tc4_call(parts, x, ln_gamma.reshape(1, D), ln_beta.reshape(1, D))
    return (xf, distances, edges, node_mask, edge_mask)
